# baseline (speedup 1.0000x reference)
import jax
import jax.numpy as jnp
from jax import lax
from jax.experimental import pallas as pl
from jax.experimental.pallas import tpu as pltpu

_B, _N, _C, _K, _TOPK = 2, 2048, 1024, 8, 2
_EPS = 1e-06
_RATIO = 0.1
_LB_W = 0.01


def _moe_kernel(tok_ref, geno_ref, Wg_ref, bg_ref, Wgg_ref, bgg_ref,
                W1_ref, b1_ref, W2_ref, b2_ref,
                out_ref, lb_ref, tbf_ref, wt_ref, cnt_ref, w1bf_ref,
                hs_ref, ws_ref):
    p = pl.program_id(0)
    b = pl.program_id(1)

    @pl.when(p == 0)
    def _gate():
        tok = tok_ref[0]
        tbf_ref[b] = tok.astype(jnp.bfloat16)
        lg = jnp.dot(tok, Wg_ref[...], preferred_element_type=jnp.float32)
        g = jnp.dot(geno_ref[0], Wgg_ref[...],
                    preferred_element_type=jnp.float32)
        lg = lg + bg_ref[...] + _RATIO * (g + bgg_ref[...])

        iota = lax.broadcasted_iota(jnp.int32, lg.shape, 1)
        m1 = jnp.max(lg, axis=1, keepdims=True)
        i1 = jnp.min(jnp.where(lg == m1, iota, _K), axis=1, keepdims=True)
        oh1 = iota == i1
        lg2 = jnp.where(oh1, jnp.float32(-1e30), lg)
        m2 = jnp.max(lg2, axis=1, keepdims=True)
        i2 = jnp.min(jnp.where(lg2 == m2, iota, _K), axis=1, keepdims=True)
        oh2 = iota == i2

        e2 = jnp.exp(m2 - m1)
        denom = 1.0 + e2
        w1 = jnp.maximum(1.0 / denom, _EPS)
        w2 = jnp.maximum(e2 / denom, _EPS)
        s = w1 + w2
        w1 = w1 / s
        w2 = w2 / s
        w = jnp.where(oh1, w1, 0.0) + jnp.where(oh2, w2, 0.0)
        wt_ref[b] = w.T

        @pl.when(b == 0)
        def _():
            cnt_ref[...] = jnp.zeros_like(cnt_ref)

        hit = oh1.astype(jnp.float32) + oh2.astype(jnp.float32)
        cnt_ref[...] += jnp.sum(hit, axis=0, keepdims=True)

        @pl.when(b == pl.num_programs(1) - 1)
        def _():
            usage = cnt_ref[...] * (1.0 / (_B * _N))
            m = jnp.mean(usage)
            v = jnp.mean((usage - m) ** 2)
            lb_ref[...] = (_LB_W * v / (m + _EPS) ** 2).reshape(1, 1)

    @pl.when((b == pl.num_programs(1) - 1) & (p < _K))
    def _cast():
        w1bf_ref[p % 2] = W1_ref[0].astype(jnp.bfloat16)

    @pl.when(p > 0)
    def _expert():
        k = p - 1
        rows = tbf_ref[b]
        h = jnp.dot(rows, w1bf_ref[(p - 1) % 2],
                    preferred_element_type=jnp.float32)
        h = jnp.maximum(h + b1_ref[0], 0.0)
        wv = wt_ref[b, k].reshape(_N, 1)
        hw = h * wv
        hacc = jnp.sum(hw.reshape(_N // 8, 8, _C), axis=0)
        hs_ref[b] = jnp.sum(hacc, axis=0)
        ws_ref[b] = jnp.full((128,), jnp.sum(wv), jnp.float32)

        @pl.when(b == pl.num_programs(1) - 1)
        def _():
            wsv = ws_ref[:, 0:1]
            num = jnp.dot(hs_ref[...], W2_ref[0],
                          preferred_element_type=jnp.float32)
            num = num + wsv * b2_ref[0]
            out_ref[0] = num / (wsv + _EPS)


@jax.jit
def kernel(tokens, geno_vec, Wg, bg, Wgg, bgg, W1, b1, W2, b2):
    B, N, C, K = _B, _N, _C, _K

    centers, lb = pl.pallas_call(
        _moe_kernel,
        grid=(K + 1, B),
        in_specs=[
            pl.BlockSpec((1, N, C), lambda p, b: (jnp.where(p == 0, b, 1), 0, 0)),
            pl.BlockSpec((1, 1, C), lambda p, b: (b, 0, 0)),
            pl.BlockSpec((C, K), lambda p, b: (0, 0)),
            pl.BlockSpec((1, K), lambda p, b: (0, 0)),
            pl.BlockSpec((C, K), lambda p, b: (0, 0)),
            pl.BlockSpec((1, K), lambda p, b: (0, 0)),
            pl.BlockSpec((1, C, C),
                         lambda p, b: (jnp.where(b == _B - 1,
                                                 jnp.minimum(p, _K - 1),
                                                 jnp.maximum(p - 1, 0)), 0, 0)),
            pl.BlockSpec((1, 1, C), lambda p, b: (jnp.maximum(p - 1, 0), 0, 0)),
            pl.BlockSpec((1, C, C), lambda p, b: (jnp.maximum(p - 1, 0), 0, 0)),
            pl.BlockSpec((1, 1, C), lambda p, b: (jnp.maximum(p - 1, 0), 0, 0)),
        ],
        out_specs=[
            pl.BlockSpec(
                (1, B, C),
                lambda p, b: ((p > 0) * (p - 1), 0, 0)),
            pl.BlockSpec((1, 1), lambda p, b: (0, 0)),
        ],
        out_shape=[
            jax.ShapeDtypeStruct((K, B, C), jnp.float32),
            jax.ShapeDtypeStruct((1, 1), jnp.float32),
        ],
        scratch_shapes=[
            pltpu.VMEM((B, N, C), jnp.bfloat16),
            pltpu.VMEM((B, K, N), jnp.float32),
            pltpu.VMEM((1, K), jnp.float32),
            pltpu.VMEM((2, C, C), jnp.bfloat16),
            pltpu.VMEM((B, C), jnp.float32),
            pltpu.VMEM((B, 128), jnp.float32),
        ],
    )(tokens, geno_vec.reshape(B, 1, C), Wg, bg.reshape(1, K),
      Wgg, bgg.reshape(1, K), W1, b1.reshape(K, 1, C), W2,
      b2.reshape(K, 1, C))

    return centers.transpose(1, 0, 2), lb.reshape(())


# speedup vs baseline: 1.0023x; 1.0023x over previous
import jax
import jax.numpy as jnp
from jax import lax
from jax.experimental import pallas as pl
from jax.experimental.pallas import tpu as pltpu

_B, _N, _C, _K, _TOPK = 2, 2048, 1024, 8, 2
_EPS = 1e-06
_RATIO = 0.1
_LB_W = 0.01


def _moe_kernel(tok_ref, geno_ref, Wg_ref, bg_ref, Wgg_ref, bgg_ref,
                W1_ref, b1_ref, W2_ref, b2_ref,
                out_ref, lb_ref, tbf_ref, wt_ref, cnt_ref, w1bf_ref,
                hs_ref, ws_ref):
    p = pl.program_id(0)
    b = pl.program_id(1)

    @pl.when(p == 0)
    def _gate():
        tok = tok_ref[0]
        tbf_ref[b] = tok.astype(jnp.bfloat16)
        lg = jnp.dot(tok, Wg_ref[...], preferred_element_type=jnp.float32)
        g = jnp.dot(geno_ref[0], Wgg_ref[...],
                    preferred_element_type=jnp.float32)
        lg = lg + bg_ref[...] + _RATIO * (g + bgg_ref[...])

        iota = lax.broadcasted_iota(jnp.int32, lg.shape, 1)
        m1 = jnp.max(lg, axis=1, keepdims=True)
        i1 = jnp.min(jnp.where(lg == m1, iota, _K), axis=1, keepdims=True)
        oh1 = iota == i1
        lg2 = jnp.where(oh1, jnp.float32(-1e30), lg)
        m2 = jnp.max(lg2, axis=1, keepdims=True)
        i2 = jnp.min(jnp.where(lg2 == m2, iota, _K), axis=1, keepdims=True)
        oh2 = iota == i2

        e2 = jnp.exp(m2 - m1)
        denom = 1.0 + e2
        w1 = jnp.maximum(1.0 / denom, _EPS)
        w2 = jnp.maximum(e2 / denom, _EPS)
        s = w1 + w2
        w1 = w1 / s
        w2 = w2 / s
        w = jnp.where(oh1, w1, 0.0) + jnp.where(oh2, w2, 0.0)
        wt_ref[b] = w.T

        @pl.when(b == 0)
        def _():
            cnt_ref[...] = jnp.zeros_like(cnt_ref)

        hit = oh1.astype(jnp.float32) + oh2.astype(jnp.float32)
        cnt_ref[...] += jnp.sum(hit, axis=0, keepdims=True)

        @pl.when(b == pl.num_programs(1) - 1)
        def _():
            usage = cnt_ref[...] * (1.0 / (_B * _N))
            m = jnp.mean(usage)
            v = jnp.mean((usage - m) ** 2)
            lb_ref[...] = (_LB_W * v / (m + _EPS) ** 2).reshape(1, 1)

    @pl.when((b == pl.num_programs(1) - 1) & (p < _K))
    def _cast():
        w1bf_ref[p % 2] = W1_ref[0].astype(jnp.bfloat16)

    @pl.when(p > 0)
    def _expert():
        k = p - 1
        rows = tbf_ref[b]
        h = jnp.dot(rows, w1bf_ref[(p - 1) % 2],
                    preferred_element_type=jnp.float32)
        h = jnp.maximum(h + b1_ref[0], 0.0)
        wv = wt_ref[b, k].reshape(_N, 1)
        hw = h * wv
        hacc = jnp.sum(hw.reshape(_N // 8, 8, _C), axis=0)
        hs_ref[b] = jnp.sum(hacc, axis=0)
        ws_ref[b] = jnp.full((128,), jnp.sum(wv), jnp.float32)

        @pl.when(b == pl.num_programs(1) - 1)
        def _():
            wsv = ws_ref[:, 0:1]
            num = jnp.dot(hs_ref[...], W2_ref[0],
                          preferred_element_type=jnp.float32)
            num = num + wsv * b2_ref[0]
            out_ref[0] = num / (wsv + _EPS)


@jax.jit
def kernel(tokens, geno_vec, Wg, bg, Wgg, bgg, W1, b1, W2, b2):
    B, N, C, K = _B, _N, _C, _K

    centers, lb = pl.pallas_call(
        _moe_kernel,
        grid=(K + 1, B),
        in_specs=[
            pl.BlockSpec((1, N, C), lambda p, b: (jnp.where(p == 0, b, 1), 0, 0)),
            pl.BlockSpec((1, 1, C), lambda p, b: (b, 0, 0)),
            pl.BlockSpec((C, K), lambda p, b: (0, 0)),
            pl.BlockSpec((1, K), lambda p, b: (0, 0)),
            pl.BlockSpec((C, K), lambda p, b: (0, 0)),
            pl.BlockSpec((1, K), lambda p, b: (0, 0)),
            pl.BlockSpec((1, C, C),
                         lambda p, b: (jnp.where(b == _B - 1,
                                                 jnp.minimum(p, _K - 1),
                                                 jnp.maximum(p - 1, 0)), 0, 0)),
            pl.BlockSpec((1, 1, C), lambda p, b: (jnp.maximum(p - 1, 0), 0, 0)),
            pl.BlockSpec((1, C, C), lambda p, b: (jnp.maximum(p - 1, 0), 0, 0)),
            pl.BlockSpec((1, 1, C), lambda p, b: (jnp.maximum(p - 1, 0), 0, 0)),
        ],
        out_specs=[
            pl.BlockSpec(
                (1, B, C),
                lambda p, b: ((p > 0) * (p - 1), 0, 0)),
            pl.BlockSpec((1, 1), lambda p, b: (0, 0)),
        ],
        out_shape=[
            jax.ShapeDtypeStruct((K, B, C), jnp.float32),
            jax.ShapeDtypeStruct((1, 1), jnp.float32),
        ],
        compiler_params=pltpu.CompilerParams(
            vmem_limit_bytes=110 * 1024 * 1024,
        ),
        scratch_shapes=[
            pltpu.VMEM((B, N, C), jnp.bfloat16),
            pltpu.VMEM((B, K, N), jnp.float32),
            pltpu.VMEM((1, K), jnp.float32),
            pltpu.VMEM((2, C, C), jnp.bfloat16),
            pltpu.VMEM((B, C), jnp.float32),
            pltpu.VMEM((B, 128), jnp.float32),
        ],
    )(tokens, geno_vec.reshape(B, 1, C), Wg, bg.reshape(1, K),
      Wgg, bgg.reshape(1, K), W1, b1.reshape(K, 1, C), W2,
      b2.reshape(K, 1, C))

    return centers.transpose(1, 0, 2), lb.reshape(())
